# baseline (speedup 1.0000x reference)
import functools

import jax
import jax.numpy as jnp
from jax import lax
from jax.experimental import pallas as pl
from jax.experimental.pallas import tpu as pltpu
from jax.experimental.pallas import tpu_sc as plsc

_N = 10000
_NP = 10240
_E = 320000
_CH = 128
_NT = 16
_BE = 64
_CHB = 32
_NCH = 10
_NB = _NCH * _CHB
_EPT = _NB * _BE
_EPAD = _NT * _EPT
_RPT = _NP // _NT
_ND = 4


def _sc_body(x_hbm, gei_hbm, sei_hbm, w_hbm, xs_hbm, y_hbm,
             ga, sa, wa, deg_v, r0, r1, r2, r3,
             deg_sh, y_sh, sem_g, sem_s):
    c = lax.axis_index("c")
    s = lax.axis_index("s")
    tid = c * _NT + s
    rows = (r0, r1, r2, r3)

    def _z16(i, _):
        deg_v[pl.ds(i * 16, 16)] = jnp.zeros((16,), jnp.float32)
        return 0
    lax.fori_loop(0, _RPT // 16, _z16, 0)
    pltpu.sync_copy(deg_v, deg_sh.at[pl.ds(s * _RPT, _RPT)])
    pltpu.sync_copy(deg_v, deg_sh.at[pl.ds(_NP + s * _RPT, _RPT)])
    plsc.subcore_barrier()

    def _deg_chunk(k, _):
        pltpu.sync_copy(gei_hbm.at[tid, pl.ds(k * _CHB, _CHB)], ga)
        pltpu.sync_copy(w_hbm.at[s, pl.ds(k * _CHB, _CHB)], wa)

        def _fire(g, _2):
            pltpu.async_copy(wa.at[g], deg_sh.at[ga.at[g]], sem_s, add=True)
            return 0
        lax.fori_loop(0, _CHB, _fire, 0)

        def _drain(g, _2):
            pltpu.make_async_copy(wa.at[g], deg_sh.at[ga.at[g]], sem_s).wait()
            return 0
        lax.fori_loop(0, _CHB, _drain, 0)
        return 0
    lax.fori_loop(0, _NCH, _deg_chunk, 0)
    plsc.subcore_barrier()

    pltpu.sync_copy(deg_sh.at[pl.ds(c * _NP + s * _RPT, _RPT)], deg_v)

    def _chunk(j, _):
        rr = s * _RPT + j * _BE
        pltpu.sync_copy(x_hbm.at[pl.ds(rr, _BE)], r0)

        def _grp(gi, _2):
            dvec = deg_v[pl.ds(j * _BE + gi * 16, 16)]
            inv16 = jnp.where(dvec == 0.0, jnp.zeros_like(dvec), 1.0 / dvec)
            for l in range(16):
                r = gi * 16 + l
                inv = inv16[l]
                for u in range(8):
                    r0[r, pl.ds(u * 16, 16)] = r0[r, pl.ds(u * 16, 16)] * inv
            return 0
        lax.fori_loop(0, _BE // 16, _grp, 0)
        pltpu.sync_copy(r0, xs_hbm.at[pl.ds(c * _NP + rr, _BE)])
        return 0
    lax.fori_loop(0, _RPT // _BE, _chunk, 0)

    def _zc(i, _):
        r0[i // 8, pl.ds((i % 8) * 16, 16)] = jnp.zeros((16,), jnp.float32)
        return 0
    lax.fori_loop(0, _BE * 8, _zc, 0)

    def _zy(j, _):
        pltpu.sync_copy(r0, y_sh.at[pl.ds(s * _RPT + j * _BE, _BE)])
        return 0
    lax.fori_loop(0, _RPT // _BE, _zy, 0)
    plsc.subcore_barrier()

    def _row_chunk(k, _):
        pltpu.sync_copy(gei_hbm.at[tid, pl.ds(k * _CHB, _CHB)], ga)
        pltpu.sync_copy(sei_hbm.at[tid, pl.ds(k * _CHB, _CHB)], sa)
        for b in range(_ND - 1):
            pltpu.async_copy(xs_hbm.at[ga.at[b]], rows[b], sem_g)
        for i in range(_CHB):
            if i + _ND - 1 < _CHB:
                pltpu.async_copy(xs_hbm.at[ga.at[i + _ND - 1]],
                                 rows[(i + _ND - 1) % _ND], sem_g)
            pltpu.make_async_copy(
                xs_hbm.at[ga.at[i]], rows[i % _ND], sem_g).wait()
            pltpu.sync_copy(rows[i % _ND], y_sh.at[sa.at[i]], add=True)
        return 0
    lax.fori_loop(0, _NCH, _row_chunk, 0)
    plsc.subcore_barrier()

    pltpu.sync_copy(y_sh.at[pl.ds(s * _RPT, _RPT)],
                    y_hbm.at[pl.ds(c * _NP + s * _RPT, _RPT)])


_sc_diffuse = functools.partial(
    pl.kernel,
    out_type=(
        jax.ShapeDtypeStruct((2 * _NP, _CH), jnp.float32),
        jax.ShapeDtypeStruct((2 * _NP, _CH), jnp.float32),
    ),
    mesh=plsc.VectorSubcoreMesh(core_axis_name="c", subcore_axis_name="s"),
    scratch_types=[
        pltpu.VMEM((_CHB, _BE), jnp.int32),
        pltpu.VMEM((_CHB, _BE), jnp.int32),
        pltpu.VMEM((_CHB, _BE), jnp.float32),
        pltpu.VMEM((_RPT,), jnp.float32),
        pltpu.VMEM((_BE, _CH), jnp.float32),
        pltpu.VMEM((_BE, _CH), jnp.float32),
        pltpu.VMEM((_BE, _CH), jnp.float32),
        pltpu.VMEM((_BE, _CH), jnp.float32),
        pltpu.VMEM_SHARED((2 * _NP,), jnp.float32),
        pltpu.VMEM_SHARED((_NP, _CH), jnp.float32),
        pltpu.SemaphoreType.DMA,
        pltpu.SemaphoreType.DMA,
    ],
)(_sc_body)


def _tc_body(x_ref, yo_ref, yi_ref, a_ref, b_ref, c_ref, bias_ref, o_ref):
    g = jnp.dot(x_ref[...], a_ref[...], preferred_element_type=jnp.float32)
    g += jnp.dot(yo_ref[...], b_ref[...], preferred_element_type=jnp.float32)
    g += jnp.dot(yi_ref[...], c_ref[...], preferred_element_type=jnp.float32)
    g += bias_ref[...]
    z = jax.nn.sigmoid(g[:, :_CH])
    ht = jnp.tanh(g[:, _CH:])
    o_ref[...] = (1.0 - z) * ht


def _tc_gates(x, yo, yi, a, b, c, bias):
    mb = 1024
    grid = (_NP // mb,)
    row_spec = pl.BlockSpec((mb, _CH), lambda i: (i, 0))
    w_spec = pl.BlockSpec((_CH, 2 * _CH), lambda i: (0, 0))
    return pl.pallas_call(
        _tc_body,
        grid=grid,
        in_specs=[row_spec, row_spec, row_spec, w_spec, w_spec, w_spec,
                  pl.BlockSpec((1, 2 * _CH), lambda i: (0, 0))],
        out_specs=row_spec,
        out_shape=jax.ShapeDtypeStruct((_NP, _CH), jnp.float32),
    )(x, yo, yi, a, b, c, bias)


def kernel(X, edge_index, edge_weight, W_z, b_z, W_r, b_r, W_h, b_h):
    del W_r, b_r
    ch = X.shape[1]

    x_p = jnp.concatenate([X, jnp.zeros((_NP - _N, ch), X.dtype)], axis=0)
    epad = _EPAD - _E
    src = edge_index[0].astype(jnp.int32)
    dst = edge_index[1].astype(jnp.int32)
    pad_idx = jnp.full((epad,), _N, jnp.int32)
    src_p = jnp.concatenate([src, pad_idx])
    dst_p = jnp.concatenate([dst, pad_idx])
    gei = jnp.concatenate([src_p, dst_p + _NP]).reshape(2 * _NT, _NB, _BE)
    sei = jnp.concatenate([dst_p, src_p]).reshape(2 * _NT, _NB, _BE)
    w_p = jnp.concatenate(
        [edge_weight.astype(jnp.float32), jnp.zeros((epad,), jnp.float32)]
    ).reshape(_NT, _NB, _BE)

    _, y = _sc_diffuse(x_p, gei, sei, w_p)
    yo = y[:_NP]
    yi = y[_NP:]

    a = jnp.concatenate([W_z[0, 0, :ch] + W_z[1, 0, :ch],
                         W_h[0, 0, :ch] + W_h[1, 0, :ch]], axis=1)
    b = jnp.concatenate([W_z[0, 1, :ch], W_h[0, 1, :ch]], axis=1)
    c = jnp.concatenate([W_z[1, 1, :ch], W_h[1, 1, :ch]], axis=1)
    bias = jnp.concatenate([b_z, b_h])[None, :]

    out = _tc_gates(x_p, yo, yi, a, b, c, bias)
    return out[:_N]


# speedup vs baseline: 1.0041x; 1.0041x over previous
import functools

import jax
import jax.numpy as jnp
from jax import lax
from jax.experimental import pallas as pl
from jax.experimental.pallas import tpu as pltpu
from jax.experimental.pallas import tpu_sc as plsc

_N = 10000
_NP = 10240
_E = 320000
_CH = 128
_NT = 16
_BE = 64
_CHB = 32
_NCH = 10
_NB = _NCH * _CHB
_EPT = _NB * _BE
_EPAD = _NT * _EPT
_RPT = _NP // _NT
_ND = 4


def _sc_body(x_hbm, gei_hbm, sei_hbm, w_hbm, xs_hbm, y_hbm,
             ga, sa, wa, deg_v, r0, r1, r2, r3,
             deg_sh, y_sh, sem_g, sem_s):
    c = lax.axis_index("c")
    s = lax.axis_index("s")
    tid = c * _NT + s
    rows = (r0, r1, r2, r3)

    def _z16(i, _):
        deg_v[pl.ds(i * 16, 16)] = jnp.zeros((16,), jnp.float32)
        return 0
    lax.fori_loop(0, _RPT // 16, _z16, 0)
    pltpu.sync_copy(deg_v, deg_sh.at[pl.ds(s * _RPT, _RPT)])
    pltpu.sync_copy(deg_v, deg_sh.at[pl.ds(_NP + s * _RPT, _RPT)])
    plsc.subcore_barrier()

    def _zc(i, _):
        r0[i // 8, pl.ds((i % 8) * 16, 16)] = jnp.zeros((16,), jnp.float32)
        return 0
    lax.fori_loop(0, _BE * 8, _zc, 0)
    for j in range(_RPT // _BE):
        pltpu.async_copy(r0, y_sh.at[pl.ds(s * _RPT + j * _BE, _BE)], sem_g)

    def _deg_chunk(k, _):
        pltpu.sync_copy(gei_hbm.at[tid, pl.ds(k * _CHB, _CHB)], ga)
        pltpu.sync_copy(w_hbm.at[s, pl.ds(k * _CHB, _CHB)], wa)

        def _fire(g, _2):
            pltpu.async_copy(wa.at[g], deg_sh.at[ga.at[g]], sem_s, add=True)
            return 0
        lax.fori_loop(0, _CHB, _fire, 0)

        def _drain(g, _2):
            pltpu.make_async_copy(wa.at[g], deg_sh.at[ga.at[g]], sem_s).wait()
            return 0
        lax.fori_loop(0, _CHB, _drain, 0)
        return 0
    lax.fori_loop(0, _NCH, _deg_chunk, 0)
    for j in range(_RPT // _BE):
        pltpu.make_async_copy(
            r0, y_sh.at[pl.ds(s * _RPT + j * _BE, _BE)], sem_g).wait()
    plsc.subcore_barrier()

    pltpu.sync_copy(deg_sh.at[pl.ds(c * _NP + s * _RPT, _RPT)], deg_v)

    def _chunk(j, _):
        rr = s * _RPT + j * _BE
        pltpu.sync_copy(x_hbm.at[pl.ds(rr, _BE)], r0)

        def _grp(gi, _2):
            dvec = deg_v[pl.ds(j * _BE + gi * 16, 16)]
            inv16 = jnp.where(dvec == 0.0, jnp.zeros_like(dvec), 1.0 / dvec)
            for l in range(16):
                r = gi * 16 + l
                inv = inv16[l]
                for u in range(8):
                    r0[r, pl.ds(u * 16, 16)] = r0[r, pl.ds(u * 16, 16)] * inv
            return 0
        lax.fori_loop(0, _BE // 16, _grp, 0)
        pltpu.sync_copy(r0, xs_hbm.at[pl.ds(c * _NP + rr, _BE)])
        return 0
    lax.fori_loop(0, _RPT // _BE, _chunk, 0)

    plsc.subcore_barrier()

    def _row_chunk(k, _):
        pltpu.sync_copy(gei_hbm.at[tid, pl.ds(k * _CHB, _CHB)], ga)
        pltpu.sync_copy(sei_hbm.at[tid, pl.ds(k * _CHB, _CHB)], sa)
        for b in range(_ND - 1):
            pltpu.async_copy(xs_hbm.at[ga.at[b]], rows[b], sem_g)
        for i in range(_CHB):
            if i + _ND - 1 < _CHB:
                pltpu.async_copy(xs_hbm.at[ga.at[i + _ND - 1]],
                                 rows[(i + _ND - 1) % _ND], sem_g)
            pltpu.make_async_copy(
                xs_hbm.at[ga.at[i]], rows[i % _ND], sem_g).wait()
            pltpu.sync_copy(rows[i % _ND], y_sh.at[sa.at[i]], add=True)
        return 0
    lax.fori_loop(0, _NCH, _row_chunk, 0)
    plsc.subcore_barrier()

    pltpu.sync_copy(y_sh.at[pl.ds(s * _RPT, _RPT)],
                    y_hbm.at[pl.ds(c * _NP + s * _RPT, _RPT)])


_sc_diffuse = functools.partial(
    pl.kernel,
    out_type=(
        jax.ShapeDtypeStruct((2 * _NP, _CH), jnp.float32),
        jax.ShapeDtypeStruct((2 * _NP, _CH), jnp.float32),
    ),
    mesh=plsc.VectorSubcoreMesh(core_axis_name="c", subcore_axis_name="s"),
    scratch_types=[
        pltpu.VMEM((_CHB, _BE), jnp.int32),
        pltpu.VMEM((_CHB, _BE), jnp.int32),
        pltpu.VMEM((_CHB, _BE), jnp.float32),
        pltpu.VMEM((_RPT,), jnp.float32),
        pltpu.VMEM((_BE, _CH), jnp.float32),
        pltpu.VMEM((_BE, _CH), jnp.float32),
        pltpu.VMEM((_BE, _CH), jnp.float32),
        pltpu.VMEM((_BE, _CH), jnp.float32),
        pltpu.VMEM_SHARED((2 * _NP,), jnp.float32),
        pltpu.VMEM_SHARED((_NP, _CH), jnp.float32),
        pltpu.SemaphoreType.DMA,
        pltpu.SemaphoreType.DMA,
    ],
)(_sc_body)


def _tc_body(x_ref, yo_ref, yi_ref, a_ref, b_ref, c_ref, bias_ref, o_ref):
    g = jnp.dot(x_ref[...], a_ref[...], preferred_element_type=jnp.float32)
    g += jnp.dot(yo_ref[...], b_ref[...], preferred_element_type=jnp.float32)
    g += jnp.dot(yi_ref[...], c_ref[...], preferred_element_type=jnp.float32)
    g += bias_ref[...]
    z = jax.nn.sigmoid(g[:, :_CH])
    ht = jnp.tanh(g[:, _CH:])
    o_ref[...] = (1.0 - z) * ht


def _tc_gates(x, yo, yi, a, b, c, bias):
    mb = 1024
    grid = (_NP // mb,)
    row_spec = pl.BlockSpec((mb, _CH), lambda i: (i, 0))
    w_spec = pl.BlockSpec((_CH, 2 * _CH), lambda i: (0, 0))
    return pl.pallas_call(
        _tc_body,
        grid=grid,
        in_specs=[row_spec, row_spec, row_spec, w_spec, w_spec, w_spec,
                  pl.BlockSpec((1, 2 * _CH), lambda i: (0, 0))],
        out_specs=row_spec,
        out_shape=jax.ShapeDtypeStruct((_NP, _CH), jnp.float32),
    )(x, yo, yi, a, b, c, bias)


def kernel(X, edge_index, edge_weight, W_z, b_z, W_r, b_r, W_h, b_h):
    del W_r, b_r
    ch = X.shape[1]

    x_p = jnp.concatenate([X, jnp.zeros((_NP - _N, ch), X.dtype)], axis=0)
    epad = _EPAD - _E
    src = edge_index[0].astype(jnp.int32)
    dst = edge_index[1].astype(jnp.int32)
    pad_idx = jnp.full((epad,), _N, jnp.int32)
    src_p = jnp.concatenate([src, pad_idx])
    dst_p = jnp.concatenate([dst, pad_idx])
    gei = jnp.concatenate([src_p, dst_p + _NP]).reshape(2 * _NT, _NB, _BE)
    sei = jnp.concatenate([dst_p, src_p]).reshape(2 * _NT, _NB, _BE)
    w_p = jnp.concatenate(
        [edge_weight.astype(jnp.float32), jnp.zeros((epad,), jnp.float32)]
    ).reshape(_NT, _NB, _BE)

    _, y = _sc_diffuse(x_p, gei, sei, w_p)
    yo = y[:_NP]
    yi = y[_NP:]

    a = jnp.concatenate([W_z[0, 0, :ch] + W_z[1, 0, :ch],
                         W_h[0, 0, :ch] + W_h[1, 0, :ch]], axis=1)
    b = jnp.concatenate([W_z[0, 1, :ch], W_h[0, 1, :ch]], axis=1)
    c = jnp.concatenate([W_z[1, 1, :ch], W_h[1, 1, :ch]], axis=1)
    bias = jnp.concatenate([b_z, b_h])[None, :]

    out = _tc_gates(x_p, yo, yi, a, b, c, bias)
    return out[:_N]
